# baseline (speedup 1.0000x reference)
import functools

import jax
import jax.numpy as jnp
from jax import lax
from jax.experimental import pallas as pl
from jax.experimental.pallas import tpu as pltpu, tpu_sc as plsc

N = 50000
E = 800000
H = 64
HALF = 25000
RPAD = 25088
TRASH = 25000
NPAD = 2 * RPAD
TPC = 1568
EPT = E // 16
EPW = E // 32

_MESH = plsc.VectorSubcoreMesh(core_axis_name="c", subcore_axis_name="s",
                               num_cores=2, num_subcores=16)

F32 = jnp.float32


def _zero16(ref, n16):
    for j in range(n16):
        ref[pl.ds(j * 16, 16)] = jnp.zeros((16,), F32)


def _ones16(ref, n16):
    for j in range(n16):
        ref[pl.ds(j * 16, 16)] = jnp.full((16,), 1.0, F32)


@functools.partial(
    pl.kernel,
    out_type=[jax.ShapeDtypeStruct((NPAD,), F32),
              jax.ShapeDtypeStruct((NPAD,), F32)],
    mesh=_MESH,
    compiler_params=pltpu.CompilerParams(use_tc_tiling_on_sc=False),
    scratch_types=[
        pltpu.VMEM((640,), jnp.int32),
        pltpu.VMEM((640,), jnp.int32),
        pltpu.VMEM((5, 128), jnp.int32),
        pltpu.VMEM((5, 128), F32),
        pltpu.VMEM((80,), jnp.int32),
        pltpu.VMEM((80,), jnp.int32),
        pltpu.VMEM((80,), jnp.int32),
        pltpu.VMEM((128,), F32),
        pltpu.VMEM((80,), F32),
        pltpu.VMEM((80,), F32),
        pltpu.VMEM((16,), F32),
        pltpu.VMEM((TPC,), F32),
        pltpu.VMEM_SHARED((RPAD,), F32),
        pltpu.VMEM_SHARED((RPAD,), F32),
        pltpu.SemaphoreType.DMA,
        pltpu.SemaphoreType.DMA,
    ],
)
def _sc_deg(row_hbm, col_hbm, cnt_hbm, lc_hbm,
            rowm, colm, lrowm, eqm, rowt, colt, lrowt,
            ones128, ones80, eq80, zb, obuf, acc_c, acc_l, sem_i, sem_s):
    cid = lax.axis_index("c")
    sid = lax.axis_index("s")
    base_n = cid * HALF
    _ones16(ones128, 8)
    _ones16(ones80, 5)
    _zero16(zb, 1)

    def zloop(i, carry):
        off = sid * TPC + i * 16
        pltpu.sync_copy(zb, acc_c.at[pl.ds(off, 16)])
        pltpu.sync_copy(zb, acc_l.at[pl.ds(off, 16)])
        return carry

    lax.fori_loop(0, TPC // 16, zloop, 0)
    plsc.subcore_barrier()

    base_e = sid * EPT

    def sbody(sci, carry):
        off = base_e + sci * 640
        d1 = pltpu.async_copy(row_hbm.at[pl.ds(off, 640)], rowm, sem_i)
        d2 = pltpu.async_copy(col_hbm.at[pl.ds(off, 640)], colm, sem_i)
        d1.wait()
        d2.wait()
        for k in range(5):
            for j in range(8):
                rv = rowm[pl.ds(k * 128 + j * 16, 16)]
                cv = colm[pl.ds(k * 128 + j * 16, 16)]
                lv = rv - base_n
                oob = jnp.logical_or(lv < 0, lv >= HALF)
                lrowm[k, pl.ds(j * 16, 16)] = jnp.where(oob, TRASH, lv)
                eqm[k, pl.ds(j * 16, 16)] = jnp.where(
                    rv == cv, jnp.full((16,), 1.0, F32), jnp.zeros((16,), F32))
        dsn = []
        for k in range(5):
            dsn.append(pltpu.async_copy(ones128, acc_c.at[lrowm.at[k]],
                                        sem_s, add=True))
            dsn.append(pltpu.async_copy(eqm.at[k], acc_l.at[lrowm.at[k]],
                                        sem_s, add=True))
        for d in dsn:
            d.wait()
        return carry

    lax.fori_loop(0, 78, sbody, 0)

    offt = base_e + 78 * 640
    pltpu.sync_copy(row_hbm.at[pl.ds(offt, 80)], rowt)
    pltpu.sync_copy(col_hbm.at[pl.ds(offt, 80)], colt)
    for j in range(5):
        rv = rowt[pl.ds(j * 16, 16)]
        cv = colt[pl.ds(j * 16, 16)]
        lv = rv - base_n
        oob = jnp.logical_or(lv < 0, lv >= HALF)
        lrowt[pl.ds(j * 16, 16)] = jnp.where(oob, TRASH, lv)
        eq80[pl.ds(j * 16, 16)] = jnp.where(
            rv == cv, jnp.full((16,), 1.0, F32), jnp.zeros((16,), F32))
    pltpu.sync_copy(ones80, acc_c.at[lrowt], add=True)
    pltpu.sync_copy(eq80, acc_l.at[lrowt], add=True)

    plsc.subcore_barrier()
    out_off = cid * RPAD + sid * TPC
    pltpu.sync_copy(acc_c.at[pl.ds(sid * TPC, TPC)], obuf)
    pltpu.sync_copy(obuf, cnt_hbm.at[pl.ds(out_off, TPC)])
    pltpu.sync_copy(acc_l.at[pl.ds(sid * TPC, TPC)], obuf)
    pltpu.sync_copy(obuf, lc_hbm.at[pl.ds(out_off, TPC)])


@functools.partial(
    pl.kernel,
    out_type=jax.ShapeDtypeStruct((NPAD, H), F32),
    mesh=_MESH,
    compiler_params=pltpu.CompilerParams(use_tc_tiling_on_sc=False),
    scratch_types=[
        pltpu.VMEM((320,), jnp.int32),
        pltpu.VMEM((320,), jnp.int32),
        pltpu.VMEM((5, 64), jnp.int32),
        pltpu.VMEM((5, 64, H), F32),
        pltpu.VMEM((16, H), F32),
        pltpu.VMEM_SHARED((RPAD, H), F32),
        pltpu.SemaphoreType.DMA,
        pltpu.SemaphoreType.DMA,
        pltpu.SemaphoreType.DMA,
    ],
)
def _sc_conv(y_hbm, row_hbm, col_hbm, out_hbm,
             rowm, colm, lrowm, rows5, zb, acc, sem_i, sem_g, sem_s):
    cid = lax.axis_index("c")
    sid = lax.axis_index("s")
    base_n = cid * HALF
    for r in range(16):
        for k in range(4):
            zb[r, pl.ds(k * 16, 16)] = jnp.zeros((16,), F32)

    def zloop(i, carry):
        pltpu.sync_copy(zb, acc.at[pl.ds(sid * TPC + i * 16, 16)])
        return carry

    lax.fori_loop(0, TPC // 16, zloop, 0)
    plsc.subcore_barrier()

    extra = jnp.where(sid < 4, 1, 0)
    nsc = 156 + extra
    base_e = (sid * 156 + jnp.minimum(sid, 4)) * 320

    def sbody(sci, carry):
        off = base_e + sci * 320
        d1 = pltpu.async_copy(row_hbm.at[pl.ds(off, 320)], rowm, sem_i)
        d2 = pltpu.async_copy(col_hbm.at[pl.ds(off, 320)], colm, sem_i)
        d1.wait()
        d2.wait()
        dg = [pltpu.async_copy(y_hbm.at[colm.at[pl.ds(k * 64, 64)]],
                               rows5.at[k], sem_g)
              for k in range(5)]
        for k in range(5):
            for j in range(4):
                rv = rowm[pl.ds(k * 64 + j * 16, 16)]
                lv = rv - base_n
                oob = jnp.logical_or(lv < 0, lv >= HALF)
                lrowm[k, pl.ds(j * 16, 16)] = jnp.where(oob, TRASH, lv)
        for d in dg:
            d.wait()
        dsn = [pltpu.async_copy(rows5.at[k], acc.at[lrowm.at[k]], sem_s,
                                add=True)
               for k in range(5)]
        for d in dsn:
            d.wait()
        return carry

    lax.fori_loop(0, nsc, sbody, 0)

    plsc.subcore_barrier()
    out_off = cid * RPAD + sid * TPC
    pltpu.sync_copy(acc.at[pl.ds(sid * TPC, TPC)],
                    out_hbm.at[pl.ds(out_off, TPC)])


@functools.partial(
    pl.kernel,
    out_type=jax.ShapeDtypeStruct((E, H), F32),
    mesh=_MESH,
    compiler_params=pltpu.CompilerParams(use_tc_tiling_on_sc=False),
    scratch_types=[
        pltpu.VMEM((640,), jnp.int32),
        pltpu.VMEM((640,), jnp.int32),
        pltpu.VMEM((5, 128, H), F32),
        pltpu.VMEM((5, 128, H), F32),
        pltpu.VMEM((40,), jnp.int32),
        pltpu.VMEM((40,), jnp.int32),
        pltpu.VMEM((40, H), F32),
        pltpu.VMEM((40, H), F32),
        pltpu.SemaphoreType.DMA,
        pltpu.SemaphoreType.DMA,
        pltpu.SemaphoreType.DMA,
    ],
)
def _sc_edge(u_hbm, v_hbm, row_hbm, col_hbm, w_hbm,
             idxrm, idxcm, bufu, bufv, idxrt, idxct, buf1t, buf2t,
             sem_i, sem_g, sem_w):
    cid = lax.axis_index("c")
    sid = lax.axis_index("s")
    wid = sid * 2 + cid
    base_e = wid * 24960

    def sbody(sci, carry):
        off = base_e + sci * 640
        d1 = pltpu.async_copy(row_hbm.at[pl.ds(off, 640)], idxrm, sem_i)
        d2 = pltpu.async_copy(col_hbm.at[pl.ds(off, 640)], idxcm, sem_i)
        d1.wait()
        d2.wait()
        dg = []
        for k in range(5):
            dg.append(pltpu.async_copy(
                u_hbm.at[idxrm.at[pl.ds(k * 128, 128)]], bufu.at[k], sem_g))
            dg.append(pltpu.async_copy(
                v_hbm.at[idxcm.at[pl.ds(k * 128, 128)]], bufv.at[k], sem_g))
        for d in dg:
            d.wait()

        def radd(r, c2):
            for k in range(5):
                for c in range(4):
                    bufu[k, r, pl.ds(c * 16, 16)] = (
                        bufu[k, r, pl.ds(c * 16, 16)]
                        + bufv[k, r, pl.ds(c * 16, 16)])
            return c2

        lax.fori_loop(0, 128, radd, 0)
        dw = [pltpu.async_copy(bufu.at[k], w_hbm.at[pl.ds(off + k * 128, 128)],
                               sem_w)
              for k in range(5)]
        for d in dw:
            d.wait()
        return carry

    lax.fori_loop(0, 39, sbody, 0)

    offt = 32 * 24960 + wid * 40
    pltpu.sync_copy(row_hbm.at[pl.ds(offt, 40)], idxrt)
    pltpu.sync_copy(col_hbm.at[pl.ds(offt, 40)], idxct)
    pltpu.async_copy(u_hbm.at[idxrt], buf1t, sem_g).wait()
    pltpu.async_copy(v_hbm.at[idxct], buf2t, sem_g).wait()

    def raddt(r, c2):
        for k in range(4):
            buf1t[r, pl.ds(k * 16, 16)] = (buf1t[r, pl.ds(k * 16, 16)]
                                           + buf2t[r, pl.ds(k * 16, 16)])
        return c2

    lax.fori_loop(0, 40, raddt, 0)
    pltpu.sync_copy(buf1t, w_hbm.at[pl.ds(offt, 40)])


def _prep_body(cnt_ref, lc_ref, dis_ref, lw_ref):
    cnt = cnt_ref[...]
    lc = lc_ref[...]
    lw = jnp.where(lc > 0.0, 0.0, 1.0)
    deg = cnt + lw
    dis = jnp.where(deg > 0.0, lax.rsqrt(jnp.maximum(deg, 1e-12)), 0.0)
    dis_ref[...] = dis
    lw_ref[...] = lw


def _tc_prep(cnt392, lc392):
    return pl.pallas_call(
        _prep_body,
        out_shape=(jax.ShapeDtypeStruct((392, 128), F32),
                   jax.ShapeDtypeStruct((392, 128), F32)),
    )(cnt392, lc392)


def _y1_body(x_ref, w_ref, dis_ref, y_ref):
    y_ref[...] = jnp.dot(x_ref[...], w_ref[...],
                         preferred_element_type=F32) * dis_ref[...]


def _tc_y1(xp, W1p, dis_n):
    blk = 1000
    return pl.pallas_call(
        _y1_body,
        grid=(N // blk,),
        in_specs=[pl.BlockSpec((blk, 128), lambda i: (i, 0)),
                  pl.BlockSpec((128, H), lambda i: (0, 0)),
                  pl.BlockSpec((blk, 1), lambda i: (i, 0))],
        out_specs=pl.BlockSpec((blk, H), lambda i: (i, 0)),
        out_shape=jax.ShapeDtypeStruct((N, H), F32),
    )(xp, W1p, dis_n)


def _fuse1_body(s_ref, y_ref, dis_ref, lw_ref, b_ref, w2_ref, h_ref, y2_ref):
    dis = dis_ref[...]
    h = jnp.maximum(dis * (s_ref[...] + lw_ref[...] * y_ref[...]) + b_ref[...], 0.0)
    h_ref[...] = h
    y2_ref[...] = jnp.dot(h, w2_ref[...], preferred_element_type=F32) * dis


def _tc_fuse1(s1, y1, dis_n, lw_n, b1r, W2):
    blk = 1000
    return pl.pallas_call(
        _fuse1_body,
        grid=(N // blk,),
        in_specs=[pl.BlockSpec((blk, H), lambda i: (i, 0)),
                  pl.BlockSpec((blk, H), lambda i: (i, 0)),
                  pl.BlockSpec((blk, 1), lambda i: (i, 0)),
                  pl.BlockSpec((blk, 1), lambda i: (i, 0)),
                  pl.BlockSpec((1, H), lambda i: (0, 0)),
                  pl.BlockSpec((H, H), lambda i: (0, 0))],
        out_specs=(pl.BlockSpec((blk, H), lambda i: (i, 0)),
                   pl.BlockSpec((blk, H), lambda i: (i, 0))),
        out_shape=(jax.ShapeDtypeStruct((N, H), F32),
                   jax.ShapeDtypeStruct((N, H), F32)),
    )(s1, y1, dis_n, lw_n, b1r, W2)


def _sigm(x):
    return 1.0 / (1.0 + jnp.exp(-x))


def _big_body(s2_ref, y2_ref, h1_ref, dis_ref, lw_ref, b2_ref,
              gw_ref, gb_ref, wf_ref, wb_ref,
              wlin_ref, blin_ref, wn1_ref, bn1_ref, wn2_ref, bn2_ref,
              we1a_ref, we1b_ref,
              nout_ref, u_ref, v_ref):
    dis = dis_ref[...]
    h1 = h1_ref[...]
    h2 = jnp.maximum(dis * (s2_ref[...] + lw_ref[...] * y2_ref[...]) + b2_ref[...], 0.0)

    gw = gw_ref[...]
    gb = gb_ref[...]

    def dot(a, b):
        return jnp.dot(a, b, preferred_element_type=F32)

    def cell(x, h, c, base, bbase, first):
        pre = [dot(x, gw[base + g]) + gb[bbase + g][None, :] for g in range(4)]
        if not first:
            for g in range(4):
                pre[g] = pre[g] + dot(h, gw[base + 4 + g])
        i = _sigm(pre[0]); f = _sigm(pre[1])
        g_ = jnp.tanh(pre[2]); o = _sigm(pre[3])
        c2 = (c * f if not first else 0.0) + i * g_
        h2_ = o * jnp.tanh(c2)
        return h2_, c2

    hf1, cf1 = cell(h1, None, None, 0, 0, True)
    hf2, _ = cell(h2, hf1, cf1, 0, 0, False)
    hb1, cb1 = cell(h2, None, None, 8, 4, True)
    hb2, _ = cell(h1, hb1, cb1, 8, 4, False)
    wf = wf_ref[...]
    wb = wb_ref[...]
    a0 = jnp.sum(hf1 * wf, axis=1, keepdims=True) + jnp.sum(hb2 * wb, axis=1, keepdims=True)
    a1 = jnp.sum(hf2 * wf, axis=1, keepdims=True) + jnp.sum(hb1 * wb, axis=1, keepdims=True)
    m = jnp.maximum(a0, a1)
    e0 = jnp.exp(a0 - m)
    e1 = jnp.exp(a1 - m)
    w0 = e0 / (e0 + e1)
    w1 = e1 / (e0 + e1)
    jk = w0 * h1 + w1 * h2
    hfin = dot(jk, wlin_ref[...]) + blin_ref[...]
    t = jnp.maximum(dot(hfin, wn1_ref[...]) + bn1_ref[...], 0.0)
    nlog = dot(t, wn2_ref[...]) + bn2_ref[...]
    nm = jnp.max(nlog, axis=1, keepdims=True)
    ne = jnp.exp(nlog - nm)
    nout_ref[...] = (ne / jnp.sum(ne, axis=1, keepdims=True))[:, :2]
    u_ref[...] = dot(hfin, we1a_ref[...])
    v_ref[...] = dot(hfin, we1b_ref[...])


def _tc_big(s2, y2, h1, dis_n, lw_n, b2r, gw, gb, wf, wb,
            Wlin, blinr, Wn1, bn1r, Wn2p, bn2p, We1a, We1b):
    blk = 1000
    full = lambda shape: pl.BlockSpec(shape, lambda i: tuple(0 for _ in shape))
    row = lambda w: pl.BlockSpec((blk, w), lambda i: (i, 0))
    return pl.pallas_call(
        _big_body,
        grid=(N // blk,),
        in_specs=[row(H), row(H), row(H), row(1), row(1), full((1, H)),
                  full((16, H, H)), full((8, H)), full((1, H)), full((1, H)),
                  full((H, H)), full((1, H)), full((H, H)), full((1, H)),
                  full((H, 8)), full((1, 8)),
                  full((H, H)), full((H, H))],
        out_specs=(pl.BlockSpec((blk, 2), lambda i: (i, 0)), row(H), row(H)),
        out_shape=(jax.ShapeDtypeStruct((N, 2), F32),
                   jax.ShapeDtypeStruct((N, H), F32),
                   jax.ShapeDtypeStruct((N, H), F32)),
    )(s2, y2, h1, dis_n, lw_n, b2r, gw, gb, wf, wb,
      Wlin, blinr, Wn1, bn1r, Wn2p, bn2p, We1a, We1b)


def _edge_body(w_ref, be1_ref, we2_ref, be2_ref, out_ref):
    t = jnp.maximum(w_ref[...] + be1_ref[...], 0.0)
    lo = jnp.dot(t, we2_ref[...], preferred_element_type=F32) + be2_ref[...]
    m = jnp.max(lo, axis=1, keepdims=True)
    e = jnp.exp(lo - m)
    out_ref[...] = (e / jnp.sum(e, axis=1, keepdims=True))[:, :2]


def _tc_edge(w, be1r, We2p, be2p):
    blk = 2000
    return pl.pallas_call(
        _edge_body,
        grid=(E // blk,),
        in_specs=[pl.BlockSpec((blk, H), lambda i: (i, 0)),
                  pl.BlockSpec((1, H), lambda i: (0, 0)),
                  pl.BlockSpec((H, 8), lambda i: (0, 0)),
                  pl.BlockSpec((1, 8), lambda i: (0, 0))],
        out_specs=pl.BlockSpec((blk, 2), lambda i: (i, 0)),
        out_shape=jax.ShapeDtypeStruct((E, 2), F32),
    )(w, be1r, We2p, be2p)


def _unpad1(a):
    return jnp.concatenate([a[:HALF], a[RPAD:RPAD + HALF]], axis=0)


def kernel(x, edge_index, W1, b1, W2, b2, Wih_f, Whh_f, bih_f, bhh_f,
           Wih_b, Whh_b, bih_b, bhh_b, Watt, batt, Wlin, blin,
           Wn1, bn1, Wn2, bn2, We1, be1, We2, be2):
    row0 = edge_index[0]
    col0 = edge_index[1]

    cnt, lc = _sc_deg(row0, col0)
    dis392, lw392 = _tc_prep(cnt.reshape(392, 128), lc.reshape(392, 128))
    dis_n = _unpad1(dis392.reshape(-1))[:, None]
    lw_n = _unpad1(lw392.reshape(-1))[:, None]

    xp = jnp.pad(x, ((0, 0), (0, 128 - x.shape[1])))
    W1p = jnp.pad(W1, ((0, 128 - W1.shape[0]), (0, 0)))
    y1 = _tc_y1(xp, W1p, dis_n)
    s1 = _unpad1(_sc_conv(y1, row0, col0))
    h1, y2 = _tc_fuse1(s1, y1, dis_n, lw_n, b1[None, :], W2)

    s2 = _unpad1(_sc_conv(y2, row0, col0))

    def gates_x(Wih):
        return [Wih[g * H:(g + 1) * H, :].T for g in range(4)]

    gw = jnp.stack(gates_x(Wih_f) + gates_x(Whh_f)
                   + gates_x(Wih_b) + gates_x(Whh_b))
    gbf = (bih_f + bhh_f).reshape(4, H)
    gbb = (bih_b + bhh_b).reshape(4, H)
    gb = jnp.concatenate([gbf, gbb], axis=0)
    wf = Watt[:H, 0][None, :]
    wb = Watt[H:, 0][None, :]
    NEG = jnp.float32(-1e30)
    Wn2p = jnp.pad(Wn2, ((0, 0), (0, 6)))
    bn2p = jnp.concatenate([bn2, jnp.full((6,), NEG)])[None, :]
    We1a = We1[:H, :]
    We1b = We1[H:, :]
    nout8, u, v = _tc_big(s2, y2, h1, dis_n, lw_n, b2[None, :],
                          gw, gb, wf, wb, Wlin, blin[None, :],
                          Wn1, bn1[None, :], Wn2p, bn2p, We1a, We1b)

    w = _sc_edge(u, v, row0, col0)
    We2p = jnp.pad(We2, ((0, 0), (0, 6)))
    be2p = jnp.concatenate([be2, jnp.full((6,), NEG)])[None, :]
    e8 = _tc_edge(w, be1[None, :], We2p, be2p)

    return nout8, e8


# speedup vs baseline: 1.2344x; 1.2344x over previous
import functools

import jax
import jax.numpy as jnp
from jax import lax
from jax.experimental import pallas as pl
from jax.experimental.pallas import tpu as pltpu, tpu_sc as plsc

N = 50000
E = 800000
H = 64
HALF = 25000
RPAD = 25088
TRASH = 25000
NPAD = 2 * RPAD
TPC = 1568
EPT = E // 16
EPW = E // 32

_MESH = plsc.VectorSubcoreMesh(core_axis_name="c", subcore_axis_name="s",
                               num_cores=2, num_subcores=16)

F32 = jnp.float32


def _zero16(ref, n16):
    for j in range(n16):
        ref[pl.ds(j * 16, 16)] = jnp.zeros((16,), F32)


def _ones16(ref, n16):
    for j in range(n16):
        ref[pl.ds(j * 16, 16)] = jnp.full((16,), 1.0, F32)


@functools.partial(
    pl.kernel,
    out_type=[jax.ShapeDtypeStruct((2 * NPAD,), F32),
              jax.ShapeDtypeStruct((2 * NPAD,), F32)],
    mesh=_MESH,
    compiler_params=pltpu.CompilerParams(use_tc_tiling_on_sc=False),
    scratch_types=[
        pltpu.VMEM((640,), jnp.int32),
        pltpu.VMEM((640,), jnp.int32),
        pltpu.VMEM((5, 128), jnp.int32),
        pltpu.VMEM((5, 128), F32),
        pltpu.VMEM((40,), jnp.int32),
        pltpu.VMEM((40,), jnp.int32),
        pltpu.VMEM((40,), jnp.int32),
        pltpu.VMEM((128,), F32),
        pltpu.VMEM((40,), F32),
        pltpu.VMEM((40,), F32),
        pltpu.VMEM((16,), F32),
        pltpu.VMEM((NPAD // 16,), F32),
        pltpu.VMEM_SHARED((NPAD,), F32),
        pltpu.VMEM_SHARED((NPAD,), F32),
        pltpu.SemaphoreType.DMA,
        pltpu.SemaphoreType.DMA,
    ],
)
def _sc_deg(row_hbm, col_hbm, cnt_hbm, lc_hbm,
            rowm, colm, lrowm, eqm, rowt, colt, lrowt,
            ones128, ones40, eq40, zb, obuf, acc_c, acc_l, sem_i, sem_s):
    cid = lax.axis_index("c")
    sid = lax.axis_index("s")
    _ones16(ones128, 8)
    _ones16(ones40, 2)
    ones40[pl.ds(24, 16)] = jnp.full((16,), 1.0, F32)
    _zero16(zb, 1)
    TPN = NPAD // 16

    def zloop(i, carry):
        off = sid * TPN + i * 16
        pltpu.sync_copy(zb, acc_c.at[pl.ds(off, 16)])
        pltpu.sync_copy(zb, acc_l.at[pl.ds(off, 16)])
        return carry

    lax.fori_loop(0, TPN // 16, zloop, 0)
    plsc.subcore_barrier()

    base_e = cid * (E // 2) + sid * 25000

    def eq_math(rsrc, csrc, ldst, edst, base, n16):
        for j in range(n16):
            rv = rsrc[pl.ds(base + j * 16, 16)]
            cv = csrc[pl.ds(base + j * 16, 16)]
            ldst[pl.ds(j * 16, 16)] = rv
            edst[pl.ds(j * 16, 16)] = jnp.where(
                rv == cv, jnp.full((16,), 1.0, F32), jnp.zeros((16,), F32))

    def sbody(sci, carry):
        off = base_e + sci * 640
        d1 = pltpu.async_copy(row_hbm.at[pl.ds(off, 640)], rowm, sem_i)
        d2 = pltpu.async_copy(col_hbm.at[pl.ds(off, 640)], colm, sem_i)
        d1.wait()
        d2.wait()
        for k in range(5):
            eq_math(rowm, colm, lrowm.at[k], eqm.at[k], k * 128, 8)
        dsn = []
        for k in range(5):
            dsn.append(pltpu.async_copy(ones128, acc_c.at[lrowm.at[k]],
                                        sem_s, add=True))
            dsn.append(pltpu.async_copy(eqm.at[k], acc_l.at[lrowm.at[k]],
                                        sem_s, add=True))
        for d in dsn:
            d.wait()
        return carry

    lax.fori_loop(0, 39, sbody, 0)

    offt = base_e + 39 * 640
    pltpu.sync_copy(row_hbm.at[pl.ds(offt, 40)], rowt)
    pltpu.sync_copy(col_hbm.at[pl.ds(offt, 40)], colt)
    for j in range(2):
        rv = rowt[pl.ds(j * 16, 16)]
        cv = colt[pl.ds(j * 16, 16)]
        lrowt[pl.ds(j * 16, 16)] = rv
        eq40[pl.ds(j * 16, 16)] = jnp.where(
            rv == cv, jnp.full((16,), 1.0, F32), jnp.zeros((16,), F32))
    rv = rowt[pl.ds(24, 16)]
    cv = colt[pl.ds(24, 16)]
    lrowt[pl.ds(24, 16)] = rv
    eq40[pl.ds(24, 16)] = jnp.where(
        rv == cv, jnp.full((16,), 1.0, F32), jnp.zeros((16,), F32))
    pltpu.sync_copy(ones40, acc_c.at[lrowt], add=True)
    pltpu.sync_copy(eq40, acc_l.at[lrowt], add=True)

    plsc.subcore_barrier()
    out_off = cid * NPAD + sid * TPN
    pltpu.sync_copy(acc_c.at[pl.ds(sid * TPN, TPN)], obuf)
    pltpu.sync_copy(obuf, cnt_hbm.at[pl.ds(out_off, TPN)])
    pltpu.sync_copy(acc_l.at[pl.ds(sid * TPN, TPN)], obuf)
    pltpu.sync_copy(obuf, lc_hbm.at[pl.ds(out_off, TPN)])


@functools.partial(
    pl.kernel,
    out_type=jax.ShapeDtypeStruct((NPAD, H), F32),
    mesh=_MESH,
    compiler_params=pltpu.CompilerParams(use_tc_tiling_on_sc=False),
    scratch_types=[
        pltpu.VMEM((320,), jnp.int32),
        pltpu.VMEM((320,), jnp.int32),
        pltpu.VMEM((5, 64), jnp.int32),
        pltpu.VMEM((5, 64, H), F32),
        pltpu.VMEM((16, H), F32),
        pltpu.VMEM_SHARED((RPAD, H), F32),
        pltpu.SemaphoreType.DMA,
        pltpu.SemaphoreType.DMA,
        pltpu.SemaphoreType.DMA,
    ],
)
def _sc_conv(y_hbm, row_hbm, col_hbm, out_hbm,
             rowm, colm, lrowm, rows5, zb, acc, sem_i, sem_g, sem_s):
    cid = lax.axis_index("c")
    sid = lax.axis_index("s")
    base_n = cid * HALF
    for r in range(16):
        for k in range(4):
            zb[r, pl.ds(k * 16, 16)] = jnp.zeros((16,), F32)

    def zloop(i, carry):
        pltpu.sync_copy(zb, acc.at[pl.ds(sid * TPC + i * 16, 16)])
        return carry

    lax.fori_loop(0, TPC // 16, zloop, 0)
    plsc.subcore_barrier()

    extra = jnp.where(sid < 4, 1, 0)
    nsc = 156 + extra
    base_e = (sid * 156 + jnp.minimum(sid, 4)) * 320

    def sbody(sci, carry):
        off = base_e + sci * 320
        d1 = pltpu.async_copy(row_hbm.at[pl.ds(off, 320)], rowm, sem_i)
        d2 = pltpu.async_copy(col_hbm.at[pl.ds(off, 320)], colm, sem_i)
        d1.wait()
        d2.wait()
        dg = [pltpu.async_copy(y_hbm.at[colm.at[pl.ds(k * 64, 64)]],
                               rows5.at[k], sem_g)
              for k in range(5)]
        for k in range(5):
            for j in range(4):
                rv = rowm[pl.ds(k * 64 + j * 16, 16)]
                lv = rv - base_n
                oob = jnp.logical_or(lv < 0, lv >= HALF)
                lrowm[k, pl.ds(j * 16, 16)] = jnp.where(oob, TRASH, lv)
        for d in dg:
            d.wait()
        dsn = [pltpu.async_copy(rows5.at[k], acc.at[lrowm.at[k]], sem_s,
                                add=True)
               for k in range(5)]
        for d in dsn:
            d.wait()
        return carry

    lax.fori_loop(0, nsc, sbody, 0)

    plsc.subcore_barrier()
    out_off = cid * RPAD + sid * TPC
    pltpu.sync_copy(acc.at[pl.ds(sid * TPC, TPC)],
                    out_hbm.at[pl.ds(out_off, TPC)])


@functools.partial(
    pl.kernel,
    out_type=jax.ShapeDtypeStruct((E, H), F32),
    mesh=_MESH,
    compiler_params=pltpu.CompilerParams(use_tc_tiling_on_sc=False),
    scratch_types=[
        pltpu.VMEM((640,), jnp.int32),
        pltpu.VMEM((640,), jnp.int32),
        pltpu.VMEM((5, 128, H), F32),
        pltpu.VMEM((5, 128, H), F32),
        pltpu.VMEM((40,), jnp.int32),
        pltpu.VMEM((40,), jnp.int32),
        pltpu.VMEM((40, H), F32),
        pltpu.VMEM((40, H), F32),
        pltpu.SemaphoreType.DMA,
        pltpu.SemaphoreType.DMA,
        pltpu.SemaphoreType.DMA,
    ],
)
def _sc_edge(u_hbm, v_hbm, row_hbm, col_hbm, w_hbm,
             idxrm, idxcm, bufu, bufv, idxrt, idxct, buf1t, buf2t,
             sem_i, sem_g, sem_w):
    cid = lax.axis_index("c")
    sid = lax.axis_index("s")
    wid = sid * 2 + cid
    base_e = wid * 24960

    def sbody(sci, carry):
        off = base_e + sci * 640
        d1 = pltpu.async_copy(row_hbm.at[pl.ds(off, 640)], idxrm, sem_i)
        d2 = pltpu.async_copy(col_hbm.at[pl.ds(off, 640)], idxcm, sem_i)
        d1.wait()
        d2.wait()
        dg = []
        for k in range(5):
            dg.append(pltpu.async_copy(
                u_hbm.at[idxrm.at[pl.ds(k * 128, 128)]], bufu.at[k], sem_g))
            dg.append(pltpu.async_copy(
                v_hbm.at[idxcm.at[pl.ds(k * 128, 128)]], bufv.at[k], sem_g))
        for d in dg:
            d.wait()

        def radd(r, c2):
            for k in range(5):
                for c in range(4):
                    bufu[k, r, pl.ds(c * 16, 16)] = (
                        bufu[k, r, pl.ds(c * 16, 16)]
                        + bufv[k, r, pl.ds(c * 16, 16)])
            return c2

        lax.fori_loop(0, 128, radd, 0)
        dw = [pltpu.async_copy(bufu.at[k], w_hbm.at[pl.ds(off + k * 128, 128)],
                               sem_w)
              for k in range(5)]
        for d in dw:
            d.wait()
        return carry

    lax.fori_loop(0, 39, sbody, 0)

    offt = 32 * 24960 + wid * 40
    pltpu.sync_copy(row_hbm.at[pl.ds(offt, 40)], idxrt)
    pltpu.sync_copy(col_hbm.at[pl.ds(offt, 40)], idxct)
    pltpu.async_copy(u_hbm.at[idxrt], buf1t, sem_g).wait()
    pltpu.async_copy(v_hbm.at[idxct], buf2t, sem_g).wait()

    def raddt(r, c2):
        for k in range(4):
            buf1t[r, pl.ds(k * 16, 16)] = (buf1t[r, pl.ds(k * 16, 16)]
                                           + buf2t[r, pl.ds(k * 16, 16)])
        return c2

    lax.fori_loop(0, 40, raddt, 0)
    pltpu.sync_copy(buf1t, w_hbm.at[pl.ds(offt, 40)])


def _prep_body(cnt_ref, lc_ref, dis_ref, lw_ref):
    cnt = cnt_ref[0] + cnt_ref[1]
    lc = lc_ref[0] + lc_ref[1]
    lw = jnp.where(lc > 0.0, 0.0, 1.0)
    deg = cnt + lw
    dis = jnp.where(deg > 0.0, lax.rsqrt(jnp.maximum(deg, 1e-12)), 0.0)
    dis_ref[...] = dis
    lw_ref[...] = lw


def _tc_prep(cnt2, lc2):
    return pl.pallas_call(
        _prep_body,
        out_shape=(jax.ShapeDtypeStruct((392, 128), F32),
                   jax.ShapeDtypeStruct((392, 128), F32)),
    )(cnt2, lc2)


def _y1_body(x_ref, w_ref, dis_ref, y_ref):
    y_ref[...] = jnp.dot(x_ref[...], w_ref[...],
                         preferred_element_type=F32) * dis_ref[...]


def _tc_y1(xp, W1p, dis_n):
    blk = 1000
    return pl.pallas_call(
        _y1_body,
        grid=(N // blk,),
        in_specs=[pl.BlockSpec((blk, 128), lambda i: (i, 0)),
                  pl.BlockSpec((128, H), lambda i: (0, 0)),
                  pl.BlockSpec((blk, 1), lambda i: (i, 0))],
        out_specs=pl.BlockSpec((blk, H), lambda i: (i, 0)),
        out_shape=jax.ShapeDtypeStruct((N, H), F32),
    )(xp, W1p, dis_n)


def _fuse1_body(s_ref, y_ref, dis_ref, lw_ref, b_ref, w2_ref, h_ref, y2_ref):
    dis = dis_ref[...]
    h = jnp.maximum(dis * (s_ref[...] + lw_ref[...] * y_ref[...]) + b_ref[...], 0.0)
    h_ref[...] = h
    y2_ref[...] = jnp.dot(h, w2_ref[...], preferred_element_type=F32) * dis


def _tc_fuse1(s1, y1, dis_n, lw_n, b1r, W2):
    blk = 1000
    return pl.pallas_call(
        _fuse1_body,
        grid=(N // blk,),
        in_specs=[pl.BlockSpec((blk, H), lambda i: (i, 0)),
                  pl.BlockSpec((blk, H), lambda i: (i, 0)),
                  pl.BlockSpec((blk, 1), lambda i: (i, 0)),
                  pl.BlockSpec((blk, 1), lambda i: (i, 0)),
                  pl.BlockSpec((1, H), lambda i: (0, 0)),
                  pl.BlockSpec((H, H), lambda i: (0, 0))],
        out_specs=(pl.BlockSpec((blk, H), lambda i: (i, 0)),
                   pl.BlockSpec((blk, H), lambda i: (i, 0))),
        out_shape=(jax.ShapeDtypeStruct((N, H), F32),
                   jax.ShapeDtypeStruct((N, H), F32)),
    )(s1, y1, dis_n, lw_n, b1r, W2)


def _sigm(x):
    return 1.0 / (1.0 + jnp.exp(-x))


def _big_body(s2_ref, y2_ref, h1_ref, dis_ref, lw_ref, b2_ref,
              gw_ref, gb_ref, wf_ref, wb_ref,
              wlin_ref, blin_ref, wn1_ref, bn1_ref, wn2_ref, bn2_ref,
              we1a_ref, we1b_ref,
              nout_ref, u_ref, v_ref):
    dis = dis_ref[...]
    h1 = h1_ref[...]
    h2 = jnp.maximum(dis * (s2_ref[...] + lw_ref[...] * y2_ref[...]) + b2_ref[...], 0.0)

    gw = gw_ref[...]
    gb = gb_ref[...]

    def dot(a, b):
        return jnp.dot(a, b, preferred_element_type=F32)

    def cell(x, h, c, base, bbase, first):
        pre = [dot(x, gw[base + g]) + gb[bbase + g][None, :] for g in range(4)]
        if not first:
            for g in range(4):
                pre[g] = pre[g] + dot(h, gw[base + 4 + g])
        i = _sigm(pre[0]); f = _sigm(pre[1])
        g_ = jnp.tanh(pre[2]); o = _sigm(pre[3])
        c2 = (c * f if not first else 0.0) + i * g_
        h2_ = o * jnp.tanh(c2)
        return h2_, c2

    hf1, cf1 = cell(h1, None, None, 0, 0, True)
    hf2, _ = cell(h2, hf1, cf1, 0, 0, False)
    hb1, cb1 = cell(h2, None, None, 8, 4, True)
    hb2, _ = cell(h1, hb1, cb1, 8, 4, False)
    wf = wf_ref[...]
    wb = wb_ref[...]
    a0 = jnp.sum(hf1 * wf, axis=1, keepdims=True) + jnp.sum(hb2 * wb, axis=1, keepdims=True)
    a1 = jnp.sum(hf2 * wf, axis=1, keepdims=True) + jnp.sum(hb1 * wb, axis=1, keepdims=True)
    m = jnp.maximum(a0, a1)
    e0 = jnp.exp(a0 - m)
    e1 = jnp.exp(a1 - m)
    w0 = e0 / (e0 + e1)
    w1 = e1 / (e0 + e1)
    jk = w0 * h1 + w1 * h2
    hfin = dot(jk, wlin_ref[...]) + blin_ref[...]
    t = jnp.maximum(dot(hfin, wn1_ref[...]) + bn1_ref[...], 0.0)
    nlog = dot(t, wn2_ref[...]) + bn2_ref[...]
    nm = jnp.max(nlog, axis=1, keepdims=True)
    ne = jnp.exp(nlog - nm)
    nout_ref[...] = ne / jnp.sum(ne, axis=1, keepdims=True)
    u_ref[...] = dot(hfin, we1a_ref[...])
    v_ref[...] = dot(hfin, we1b_ref[...])


def _tc_big(s2, y2, h1, dis_n, lw_n, b2r, gw, gb, wf, wb,
            Wlin, blinr, Wn1, bn1r, Wn2p, bn2p, We1a, We1b):
    blk = 1000
    full = lambda shape: pl.BlockSpec(shape, lambda i: tuple(0 for _ in shape))
    row = lambda w: pl.BlockSpec((blk, w), lambda i: (i, 0))
    return pl.pallas_call(
        _big_body,
        grid=(N // blk,),
        in_specs=[row(H), row(H), row(H), row(1), row(1), full((1, H)),
                  full((16, H, H)), full((8, H)), full((1, H)), full((1, H)),
                  full((H, H)), full((1, H)), full((H, H)), full((1, H)),
                  full((H, 8)), full((1, 8)),
                  full((H, H)), full((H, H))],
        out_specs=(pl.BlockSpec((blk, 8), lambda i: (i, 0)), row(H), row(H)),
        out_shape=(jax.ShapeDtypeStruct((N, 8), F32),
                   jax.ShapeDtypeStruct((N, H), F32),
                   jax.ShapeDtypeStruct((N, H), F32)),
    )(s2, y2, h1, dis_n, lw_n, b2r, gw, gb, wf, wb,
      Wlin, blinr, Wn1, bn1r, Wn2p, bn2p, We1a, We1b)


def _edge_body(w_ref, be1_ref, we2_ref, be2_ref, out_ref):
    t = jnp.maximum(w_ref[...] + be1_ref[...], 0.0)
    lo = jnp.dot(t, we2_ref[...], preferred_element_type=F32) + be2_ref[...]
    m = jnp.max(lo, axis=1, keepdims=True)
    e = jnp.exp(lo - m)
    out_ref[...] = e / jnp.sum(e, axis=1, keepdims=True)


def _tc_edge(w, be1r, We2p, be2p):
    blk = 2000
    return pl.pallas_call(
        _edge_body,
        grid=(E // blk,),
        in_specs=[pl.BlockSpec((blk, H), lambda i: (i, 0)),
                  pl.BlockSpec((1, H), lambda i: (0, 0)),
                  pl.BlockSpec((H, 8), lambda i: (0, 0)),
                  pl.BlockSpec((1, 8), lambda i: (0, 0))],
        out_specs=pl.BlockSpec((blk, 8), lambda i: (i, 0)),
        out_shape=jax.ShapeDtypeStruct((E, 8), F32),
    )(w, be1r, We2p, be2p)


def _unpad1(a):
    return jnp.concatenate([a[:HALF], a[RPAD:RPAD + HALF]], axis=0)


def kernel(x, edge_index, W1, b1, W2, b2, Wih_f, Whh_f, bih_f, bhh_f,
           Wih_b, Whh_b, bih_b, bhh_b, Watt, batt, Wlin, blin,
           Wn1, bn1, Wn2, bn2, We1, be1, We2, be2):
    row0 = edge_index[0]
    col0 = edge_index[1]

    cnt, lc = _sc_deg(row0, col0)
    dis392, lw392 = _tc_prep(cnt.reshape(2, 392, 128), lc.reshape(2, 392, 128))
    dis_n = dis392.reshape(-1)[:N, None]
    lw_n = lw392.reshape(-1)[:N, None]

    xp = jnp.pad(x, ((0, 0), (0, 128 - x.shape[1])))
    W1p = jnp.pad(W1, ((0, 128 - W1.shape[0]), (0, 0)))
    y1 = _tc_y1(xp, W1p, dis_n)
    s1 = _unpad1(_sc_conv(y1, row0, col0))
    h1, y2 = _tc_fuse1(s1, y1, dis_n, lw_n, b1[None, :], W2)

    s2 = _unpad1(_sc_conv(y2, row0, col0))

    def gates_x(Wih):
        return [Wih[g * H:(g + 1) * H, :].T for g in range(4)]

    gw = jnp.stack(gates_x(Wih_f) + gates_x(Whh_f)
                   + gates_x(Wih_b) + gates_x(Whh_b))
    gbf = (bih_f + bhh_f).reshape(4, H)
    gbb = (bih_b + bhh_b).reshape(4, H)
    gb = jnp.concatenate([gbf, gbb], axis=0)
    wf = Watt[:H, 0][None, :]
    wb = Watt[H:, 0][None, :]
    NEG = jnp.float32(-1e30)
    Wn2p = jnp.pad(Wn2, ((0, 0), (0, 6)))
    bn2p = jnp.concatenate([bn2, jnp.full((6,), NEG)])[None, :]
    We1a = We1[:H, :]
    We1b = We1[H:, :]
    nout8, u, v = _tc_big(s2, y2, h1, dis_n, lw_n, b2[None, :],
                          gw, gb, wf, wb, Wlin, blin[None, :],
                          Wn1, bn1[None, :], Wn2p, bn2p, We1a, We1b)

    w = _sc_edge(u, v, row0, col0)
    We2p = jnp.pad(We2, ((0, 0), (0, 6)))
    be2p = jnp.concatenate([be2, jnp.full((6,), NEG)])[None, :]
    e8 = _tc_edge(w, be1[None, :], We2p, be2p)

    return nout8[:, :2], e8[:, :2]


# speedup vs baseline: 1.2390x; 1.0037x over previous
import functools

import jax
import jax.numpy as jnp
from jax import lax
from jax.experimental import pallas as pl
from jax.experimental.pallas import tpu as pltpu, tpu_sc as plsc

N = 50000
E = 800000
H = 64
HALF = 25000
RPAD = 25088
TRASH = 25000
NPAD = 2 * RPAD
TPC = 1568
EPT = E // 16
EPW = E // 32

_MESH = plsc.VectorSubcoreMesh(core_axis_name="c", subcore_axis_name="s",
                               num_cores=2, num_subcores=16)

F32 = jnp.float32


def _zero16(ref, n16):
    for j in range(n16):
        ref[pl.ds(j * 16, 16)] = jnp.zeros((16,), F32)


def _ones16(ref, n16):
    for j in range(n16):
        ref[pl.ds(j * 16, 16)] = jnp.full((16,), 1.0, F32)


@functools.partial(
    pl.kernel,
    out_type=[jax.ShapeDtypeStruct((2 * NPAD,), F32),
              jax.ShapeDtypeStruct((2 * NPAD,), F32)],
    mesh=_MESH,
    compiler_params=pltpu.CompilerParams(use_tc_tiling_on_sc=False),
    scratch_types=[
        pltpu.VMEM((640,), jnp.int32),
        pltpu.VMEM((640,), jnp.int32),
        pltpu.VMEM((5, 128), jnp.int32),
        pltpu.VMEM((5, 128), F32),
        pltpu.VMEM((40,), jnp.int32),
        pltpu.VMEM((40,), jnp.int32),
        pltpu.VMEM((40,), jnp.int32),
        pltpu.VMEM((128,), F32),
        pltpu.VMEM((40,), F32),
        pltpu.VMEM((40,), F32),
        pltpu.VMEM((16,), F32),
        pltpu.VMEM((NPAD // 16,), F32),
        pltpu.VMEM_SHARED((NPAD,), F32),
        pltpu.VMEM_SHARED((NPAD,), F32),
        pltpu.SemaphoreType.DMA,
        pltpu.SemaphoreType.DMA,
    ],
)
def _sc_deg(row_hbm, col_hbm, cnt_hbm, lc_hbm,
            rowm, colm, lrowm, eqm, rowt, colt, lrowt,
            ones128, ones40, eq40, zb, obuf, acc_c, acc_l, sem_i, sem_s):
    cid = lax.axis_index("c")
    sid = lax.axis_index("s")
    _ones16(ones128, 8)
    _ones16(ones40, 2)
    ones40[pl.ds(24, 16)] = jnp.full((16,), 1.0, F32)
    _zero16(zb, 1)
    TPN = NPAD // 16

    def zloop(i, carry):
        off = sid * TPN + i * 16
        pltpu.sync_copy(zb, acc_c.at[pl.ds(off, 16)])
        pltpu.sync_copy(zb, acc_l.at[pl.ds(off, 16)])
        return carry

    lax.fori_loop(0, TPN // 16, zloop, 0)
    plsc.subcore_barrier()

    base_e = cid * (E // 2) + sid * 25000

    def eq_math(rsrc, csrc, ldst, edst, base, n16):
        for j in range(n16):
            rv = rsrc[pl.ds(base + j * 16, 16)]
            cv = csrc[pl.ds(base + j * 16, 16)]
            ldst[pl.ds(j * 16, 16)] = rv
            edst[pl.ds(j * 16, 16)] = jnp.where(
                rv == cv, jnp.full((16,), 1.0, F32), jnp.zeros((16,), F32))

    def sbody(sci, carry):
        off = base_e + sci * 640
        d1 = pltpu.async_copy(row_hbm.at[pl.ds(off, 640)], rowm, sem_i)
        d2 = pltpu.async_copy(col_hbm.at[pl.ds(off, 640)], colm, sem_i)
        d1.wait()
        d2.wait()
        for k in range(5):
            eq_math(rowm, colm, lrowm.at[k], eqm.at[k], k * 128, 8)
        dsn = []
        for k in range(5):
            dsn.append(pltpu.async_copy(ones128, acc_c.at[lrowm.at[k]],
                                        sem_s, add=True))
            dsn.append(pltpu.async_copy(eqm.at[k], acc_l.at[lrowm.at[k]],
                                        sem_s, add=True))
        for d in dsn:
            d.wait()
        return carry

    lax.fori_loop(0, 39, sbody, 0)

    offt = base_e + 39 * 640
    pltpu.sync_copy(row_hbm.at[pl.ds(offt, 40)], rowt)
    pltpu.sync_copy(col_hbm.at[pl.ds(offt, 40)], colt)
    for j in range(2):
        rv = rowt[pl.ds(j * 16, 16)]
        cv = colt[pl.ds(j * 16, 16)]
        lrowt[pl.ds(j * 16, 16)] = rv
        eq40[pl.ds(j * 16, 16)] = jnp.where(
            rv == cv, jnp.full((16,), 1.0, F32), jnp.zeros((16,), F32))
    rv = rowt[pl.ds(24, 16)]
    cv = colt[pl.ds(24, 16)]
    lrowt[pl.ds(24, 16)] = rv
    eq40[pl.ds(24, 16)] = jnp.where(
        rv == cv, jnp.full((16,), 1.0, F32), jnp.zeros((16,), F32))
    pltpu.sync_copy(ones40, acc_c.at[lrowt], add=True)
    pltpu.sync_copy(eq40, acc_l.at[lrowt], add=True)

    plsc.subcore_barrier()
    out_off = cid * NPAD + sid * TPN
    pltpu.sync_copy(acc_c.at[pl.ds(sid * TPN, TPN)], obuf)
    pltpu.sync_copy(obuf, cnt_hbm.at[pl.ds(out_off, TPN)])
    pltpu.sync_copy(acc_l.at[pl.ds(sid * TPN, TPN)], obuf)
    pltpu.sync_copy(obuf, lc_hbm.at[pl.ds(out_off, TPN)])


@functools.partial(
    pl.kernel,
    out_type=jax.ShapeDtypeStruct((NPAD, H), F32),
    mesh=_MESH,
    compiler_params=pltpu.CompilerParams(use_tc_tiling_on_sc=False),
    scratch_types=[
        pltpu.VMEM((320,), jnp.int32),
        pltpu.VMEM((320,), jnp.int32),
        pltpu.VMEM((5, 64), jnp.int32),
        pltpu.VMEM((5, 64, H), F32),
        pltpu.VMEM((16, H), F32),
        pltpu.VMEM_SHARED((RPAD, H), F32),
        pltpu.SemaphoreType.DMA,
        pltpu.SemaphoreType.DMA,
        pltpu.SemaphoreType.DMA,
    ],
)
def _sc_conv(y_hbm, row_hbm, col_hbm, out_hbm,
             rowm, colm, lrowm, rows5, zb, acc, sem_i, sem_g, sem_s):
    cid = lax.axis_index("c")
    sid = lax.axis_index("s")
    base_n = cid * HALF
    for r in range(16):
        for k in range(4):
            zb[r, pl.ds(k * 16, 16)] = jnp.zeros((16,), F32)

    def zloop(i, carry):
        pltpu.sync_copy(zb, acc.at[pl.ds(sid * TPC + i * 16, 16)])
        return carry

    lax.fori_loop(0, TPC // 16, zloop, 0)
    plsc.subcore_barrier()

    extra = jnp.where(sid < 4, 1, 0)
    nsc = 156 + extra
    base_e = (sid * 156 + jnp.minimum(sid, 4)) * 320

    def sbody(sci, carry):
        off = base_e + sci * 320
        d1 = pltpu.async_copy(row_hbm.at[pl.ds(off, 320)], rowm, sem_i)
        d2 = pltpu.async_copy(col_hbm.at[pl.ds(off, 320)], colm, sem_i)
        d1.wait()
        d2.wait()
        dg = [pltpu.async_copy(y_hbm.at[colm.at[pl.ds(k * 64, 64)]],
                               rows5.at[k], sem_g)
              for k in range(5)]
        for k in range(5):
            for j in range(4):
                rv = rowm[pl.ds(k * 64 + j * 16, 16)]
                lv = rv - base_n
                oob = jnp.logical_or(lv < 0, lv >= HALF)
                lrowm[k, pl.ds(j * 16, 16)] = jnp.where(oob, TRASH, lv)
        for d in dg:
            d.wait()
        dsn = [pltpu.async_copy(rows5.at[k], acc.at[lrowm.at[k]], sem_s,
                                add=True)
               for k in range(5)]
        for d in dsn:
            d.wait()
        return carry

    lax.fori_loop(0, nsc, sbody, 0)

    plsc.subcore_barrier()
    out_off = cid * RPAD + sid * TPC
    pltpu.sync_copy(acc.at[pl.ds(sid * TPC, TPC)],
                    out_hbm.at[pl.ds(out_off, TPC)])


@functools.partial(
    pl.kernel,
    out_type=jax.ShapeDtypeStruct((E, H), F32),
    mesh=_MESH,
    compiler_params=pltpu.CompilerParams(use_tc_tiling_on_sc=False),
    scratch_types=[
        pltpu.VMEM((640,), jnp.int32),
        pltpu.VMEM((640,), jnp.int32),
        pltpu.VMEM((5, 128, H), F32),
        pltpu.VMEM((5, 128, H), F32),
        pltpu.VMEM((40,), jnp.int32),
        pltpu.VMEM((40,), jnp.int32),
        pltpu.VMEM((40, H), F32),
        pltpu.VMEM((40, H), F32),
        pltpu.SemaphoreType.DMA,
        pltpu.SemaphoreType.DMA,
        pltpu.SemaphoreType.DMA,
    ],
)
def _sc_edge(u_hbm, v_hbm, row_hbm, col_hbm, w_hbm,
             idxrm, idxcm, bufu, bufv, idxrt, idxct, buf1t, buf2t,
             sem_i, sem_g, sem_w):
    cid = lax.axis_index("c")
    sid = lax.axis_index("s")
    wid = sid * 2 + cid
    base_e = wid * 24960

    def sbody(sci, carry):
        off = base_e + sci * 640
        d1 = pltpu.async_copy(row_hbm.at[pl.ds(off, 640)], idxrm, sem_i)
        d2 = pltpu.async_copy(col_hbm.at[pl.ds(off, 640)], idxcm, sem_i)
        d1.wait()
        d2.wait()
        dg = []
        for k in range(5):
            dg.append(pltpu.async_copy(
                u_hbm.at[idxrm.at[pl.ds(k * 128, 128)]], bufu.at[k], sem_g))
            dg.append(pltpu.async_copy(
                v_hbm.at[idxcm.at[pl.ds(k * 128, 128)]], bufv.at[k], sem_g))
        for d in dg:
            d.wait()

        def radd(r, c2):
            for k in range(5):
                for c in range(4):
                    bufu[k, r, pl.ds(c * 16, 16)] = (
                        bufu[k, r, pl.ds(c * 16, 16)]
                        + bufv[k, r, pl.ds(c * 16, 16)])
            return c2

        lax.fori_loop(0, 128, radd, 0)
        dw = [pltpu.async_copy(bufu.at[k], w_hbm.at[pl.ds(off + k * 128, 128)],
                               sem_w)
              for k in range(5)]
        for d in dw:
            d.wait()
        return carry

    lax.fori_loop(0, 39, sbody, 0)

    offt = 32 * 24960 + wid * 40
    pltpu.sync_copy(row_hbm.at[pl.ds(offt, 40)], idxrt)
    pltpu.sync_copy(col_hbm.at[pl.ds(offt, 40)], idxct)
    pltpu.async_copy(u_hbm.at[idxrt], buf1t, sem_g).wait()
    pltpu.async_copy(v_hbm.at[idxct], buf2t, sem_g).wait()

    def raddt(r, c2):
        for k in range(4):
            buf1t[r, pl.ds(k * 16, 16)] = (buf1t[r, pl.ds(k * 16, 16)]
                                           + buf2t[r, pl.ds(k * 16, 16)])
        return c2

    lax.fori_loop(0, 40, raddt, 0)
    pltpu.sync_copy(buf1t, w_hbm.at[pl.ds(offt, 40)])


def _prep_body(cnt_ref, lc_ref, dis_ref, lw_ref):
    cnt = cnt_ref[0] + cnt_ref[1]
    lc = lc_ref[0] + lc_ref[1]
    lw = jnp.where(lc > 0.0, 0.0, 1.0)
    deg = cnt + lw
    dis = jnp.where(deg > 0.0, lax.rsqrt(jnp.maximum(deg, 1e-12)), 0.0)
    dis_ref[...] = dis
    lw_ref[...] = lw


def _tc_prep(cnt2, lc2):
    return pl.pallas_call(
        _prep_body,
        out_shape=(jax.ShapeDtypeStruct((392, 128), F32),
                   jax.ShapeDtypeStruct((392, 128), F32)),
    )(cnt2, lc2)


def _y1_body(x_ref, w_ref, dis_ref, y_ref):
    y_ref[...] = jnp.dot(x_ref[...], w_ref[...],
                         preferred_element_type=F32) * dis_ref[...]


def _tc_y1(xp, W1p, dis_n):
    blk = 1000
    return pl.pallas_call(
        _y1_body,
        grid=(N // blk,),
        in_specs=[pl.BlockSpec((blk, 128), lambda i: (i, 0)),
                  pl.BlockSpec((128, H), lambda i: (0, 0)),
                  pl.BlockSpec((blk, 1), lambda i: (i, 0))],
        out_specs=pl.BlockSpec((blk, H), lambda i: (i, 0)),
        out_shape=jax.ShapeDtypeStruct((N, H), F32),
    )(xp, W1p, dis_n)


def _fuse1_body(s_ref, y_ref, dis_ref, lw_ref, b_ref, w2_ref, h_ref, y2_ref):
    dis = dis_ref[...]
    h = jnp.maximum(dis * (s_ref[...] + lw_ref[...] * y_ref[...]) + b_ref[...], 0.0)
    h_ref[...] = h
    y2_ref[...] = jnp.dot(h, w2_ref[...], preferred_element_type=F32) * dis


def _tc_fuse1(s1, y1, dis_n, lw_n, b1r, W2):
    blk = 1000
    return pl.pallas_call(
        _fuse1_body,
        grid=(N // blk,),
        in_specs=[pl.BlockSpec((blk, H), lambda i: (i, 0)),
                  pl.BlockSpec((blk, H), lambda i: (i, 0)),
                  pl.BlockSpec((blk, 1), lambda i: (i, 0)),
                  pl.BlockSpec((blk, 1), lambda i: (i, 0)),
                  pl.BlockSpec((1, H), lambda i: (0, 0)),
                  pl.BlockSpec((H, H), lambda i: (0, 0))],
        out_specs=(pl.BlockSpec((blk, H), lambda i: (i, 0)),
                   pl.BlockSpec((blk, H), lambda i: (i, 0))),
        out_shape=(jax.ShapeDtypeStruct((N, H), F32),
                   jax.ShapeDtypeStruct((N, H), F32)),
    )(s1, y1, dis_n, lw_n, b1r, W2)


def _sigm(x):
    return 1.0 / (1.0 + jnp.exp(-x))


def _big_body(s2_ref, y2_ref, h1_ref, dis_ref, lw_ref, b2_ref,
              gw_ref, gb_ref, wf_ref, wb_ref,
              wlin_ref, blin_ref,
              we1a_ref, we1b_ref,
              hfin_ref, u_ref, v_ref):
    dis = dis_ref[...]
    h1 = h1_ref[...]
    h2 = jnp.maximum(dis * (s2_ref[...] + lw_ref[...] * y2_ref[...]) + b2_ref[...], 0.0)

    gw = gw_ref[...]
    gb = gb_ref[...]

    def dot(a, b):
        return jnp.dot(a, b, preferred_element_type=F32)

    def cell(x, h, c, base, bbase, first):
        pre = [dot(x, gw[base + g]) + gb[bbase + g][None, :] for g in range(4)]
        if not first:
            for g in range(4):
                pre[g] = pre[g] + dot(h, gw[base + 4 + g])
        i = _sigm(pre[0]); f = _sigm(pre[1])
        g_ = jnp.tanh(pre[2]); o = _sigm(pre[3])
        c2 = (c * f if not first else 0.0) + i * g_
        h2_ = o * jnp.tanh(c2)
        return h2_, c2

    hf1, cf1 = cell(h1, None, None, 0, 0, True)
    hf2, _ = cell(h2, hf1, cf1, 0, 0, False)
    hb1, cb1 = cell(h2, None, None, 8, 4, True)
    hb2, _ = cell(h1, hb1, cb1, 8, 4, False)
    wf = wf_ref[...]
    wb = wb_ref[...]
    a0 = jnp.sum(hf1 * wf, axis=1, keepdims=True) + jnp.sum(hb2 * wb, axis=1, keepdims=True)
    a1 = jnp.sum(hf2 * wf, axis=1, keepdims=True) + jnp.sum(hb1 * wb, axis=1, keepdims=True)
    m = jnp.maximum(a0, a1)
    e0 = jnp.exp(a0 - m)
    e1 = jnp.exp(a1 - m)
    w0 = e0 / (e0 + e1)
    w1 = e1 / (e0 + e1)
    jk = w0 * h1 + w1 * h2
    hfin = dot(jk, wlin_ref[...]) + blin_ref[...]
    hfin_ref[...] = hfin
    u_ref[...] = dot(hfin, we1a_ref[...])
    v_ref[...] = dot(hfin, we1b_ref[...])


def _tc_big(s2, y2, h1, dis_n, lw_n, b2r, gw, gb, wf, wb,
            Wlin, blinr, We1a, We1b):
    blk = 1000
    full = lambda shape: pl.BlockSpec(shape, lambda i: tuple(0 for _ in shape))
    row = lambda w: pl.BlockSpec((blk, w), lambda i: (i, 0))
    return pl.pallas_call(
        _big_body,
        grid=(N // blk,),
        in_specs=[row(H), row(H), row(H), row(1), row(1), full((1, H)),
                  full((16, H, H)), full((8, H)), full((1, H)), full((1, H)),
                  full((H, H)), full((1, H)),
                  full((H, H)), full((H, H))],
        out_specs=(row(H), row(H), row(H)),
        out_shape=(jax.ShapeDtypeStruct((N, H), F32),
                   jax.ShapeDtypeStruct((N, H), F32),
                   jax.ShapeDtypeStruct((N, H), F32)),
    )(s2, y2, h1, dis_n, lw_n, b2r, gw, gb, wf, wb,
      Wlin, blinr, We1a, We1b)


def _node_body(hfin_ref, wn1_ref, bn1_ref, wn2_ref, bn2_ref, nout_ref):
    hfin = hfin_ref[...]
    t = jnp.maximum(jnp.dot(hfin, wn1_ref[...], preferred_element_type=F32)
                    + bn1_ref[...], 0.0)
    nlog = jnp.dot(t, wn2_ref[...], preferred_element_type=F32) + bn2_ref[...]
    nm = jnp.max(nlog, axis=1, keepdims=True)
    ne = jnp.exp(nlog - nm)
    nout_ref[...] = ne / jnp.sum(ne, axis=1, keepdims=True)


def _tc_node(hfin, Wn1, bn1r, Wn2p, bn2p):
    blk = 1000
    full = lambda shape: pl.BlockSpec(shape, lambda i: tuple(0 for _ in shape))
    return pl.pallas_call(
        _node_body,
        grid=(N // blk,),
        in_specs=[pl.BlockSpec((blk, H), lambda i: (i, 0)),
                  full((H, H)), full((1, H)), full((H, 8)), full((1, 8))],
        out_specs=pl.BlockSpec((blk, 8), lambda i: (i, 0)),
        out_shape=jax.ShapeDtypeStruct((N, 8), F32),
    )(hfin, Wn1, bn1r, Wn2p, bn2p)


def _edge_body(w_ref, be1_ref, we2_ref, be2_ref, out_ref):
    t = jnp.maximum(w_ref[...] + be1_ref[...], 0.0)
    lo = jnp.dot(t, we2_ref[...], preferred_element_type=F32) + be2_ref[...]
    m = jnp.max(lo, axis=1, keepdims=True)
    e = jnp.exp(lo - m)
    out_ref[...] = e / jnp.sum(e, axis=1, keepdims=True)


def _tc_edge(w, be1r, We2p, be2p):
    blk = 2000
    return pl.pallas_call(
        _edge_body,
        grid=(E // blk,),
        in_specs=[pl.BlockSpec((blk, H), lambda i: (i, 0)),
                  pl.BlockSpec((1, H), lambda i: (0, 0)),
                  pl.BlockSpec((H, 8), lambda i: (0, 0)),
                  pl.BlockSpec((1, 8), lambda i: (0, 0))],
        out_specs=pl.BlockSpec((blk, 8), lambda i: (i, 0)),
        out_shape=jax.ShapeDtypeStruct((E, 8), F32),
    )(w, be1r, We2p, be2p)


def _unpad1(a):
    return jnp.concatenate([a[:HALF], a[RPAD:RPAD + HALF]], axis=0)


def kernel(x, edge_index, W1, b1, W2, b2, Wih_f, Whh_f, bih_f, bhh_f,
           Wih_b, Whh_b, bih_b, bhh_b, Watt, batt, Wlin, blin,
           Wn1, bn1, Wn2, bn2, We1, be1, We2, be2):
    row0 = edge_index[0]
    col0 = edge_index[1]

    cnt, lc = _sc_deg(row0, col0)
    dis392, lw392 = _tc_prep(cnt.reshape(2, 392, 128), lc.reshape(2, 392, 128))
    dis_n = dis392.reshape(-1)[:N, None]
    lw_n = lw392.reshape(-1)[:N, None]

    xp = jnp.pad(x, ((0, 0), (0, 128 - x.shape[1])))
    W1p = jnp.pad(W1, ((0, 128 - W1.shape[0]), (0, 0)))
    y1 = _tc_y1(xp, W1p, dis_n)
    s1 = _unpad1(_sc_conv(y1, row0, col0))
    h1, y2 = _tc_fuse1(s1, y1, dis_n, lw_n, b1[None, :], W2)

    s2 = _unpad1(_sc_conv(y2, row0, col0))

    def gates_x(Wih):
        return [Wih[g * H:(g + 1) * H, :].T for g in range(4)]

    gw = jnp.stack(gates_x(Wih_f) + gates_x(Whh_f)
                   + gates_x(Wih_b) + gates_x(Whh_b))
    gbf = (bih_f + bhh_f).reshape(4, H)
    gbb = (bih_b + bhh_b).reshape(4, H)
    gb = jnp.concatenate([gbf, gbb], axis=0)
    wf = Watt[:H, 0][None, :]
    wb = Watt[H:, 0][None, :]
    NEG = jnp.float32(-1e30)
    Wn2p = jnp.pad(Wn2, ((0, 0), (0, 6)))
    bn2p = jnp.concatenate([bn2, jnp.full((6,), NEG)])[None, :]
    We1a = We1[:H, :]
    We1b = We1[H:, :]
    hfin, u, v = _tc_big(s2, y2, h1, dis_n, lw_n, b2[None, :],
                         gw, gb, wf, wb, Wlin, blin[None, :], We1a, We1b)

    w = _sc_edge(u, v, row0, col0)
    nout8 = _tc_node(hfin, Wn1, bn1[None, :], Wn2p, bn2p)
    We2p = jnp.pad(We2, ((0, 0), (0, 6)))
    be2p = jnp.concatenate([be2, jnp.full((6,), NEG)])[None, :]
    e8 = _tc_edge(w, be1[None, :], We2p, be2p)

    return nout8[:, :2], e8[:, :2]
